# baseline (speedup 1.0000x reference)
import functools

import jax
import jax.numpy as jnp
from jax import lax
from jax.experimental import pallas as pl
from jax.experimental.pallas import tpu as pltpu
from jax.experimental.pallas import tpu_sc as plsc

_ATOM_DIMS = (119, 4, 12, 12, 10, 6, 6, 2, 2)
_BOND_DIMS = (5, 6, 2)
_ATOM_PAD = 176
_BOND_PAD = 16
_DIM_H = 112
_DIM_EMB = 128

_NB = 1000
_EB = 16000
_G = _EB // 128
_C_ROWS = 64

_CH = 400
_NW = 32
_NCH = 25


def _prep_body(ea_ref, b0_ref, b1_ref, b2_ref, codes_ref, c_ref):
    jj = jax.lax.broadcasted_iota(jnp.int32, (384, 128), 0)
    mm = jax.lax.broadcasted_iota(jnp.int32, (384, 128), 1)
    wsel = jnp.where(jj % 3 == 0, 12.0, jnp.where(jj % 3 == 1, 2.0, 1.0))
    w = (jj // 3 == mm).astype(jnp.float32) * wsel
    f = ea_ref[...].astype(jnp.float32)
    codes_ref[...] = jnp.dot(
        f, w, preferred_element_type=jnp.float32).astype(jnp.int32)

    tbl = jnp.concatenate(
        [b0_ref[...], b1_ref[...], b2_ref[...],
         jnp.zeros((_BOND_PAD - 13, _DIM_EMB), jnp.float32)], axis=0)
    k = jax.lax.broadcasted_iota(jnp.int32, (_C_ROWS, 1), 0)
    iota = jax.lax.broadcasted_iota(jnp.int32, (_C_ROWS, _BOND_PAD), 1)
    mh = ((iota == k // 12).astype(jnp.float32)
          + (iota == 5 + (k % 12) // 2).astype(jnp.float32)
          + (iota == 11 + k % 2).astype(jnp.float32))
    c_ref[...] = jnp.dot(mh, tbl, preferred_element_type=jnp.float32)


def _node_body(x_ref, pe_ref, w1_ref, b1_ref, w2_ref, b2_ref, *rest):
    atab_refs, out_ref = rest[:9], rest[9]
    tbl = jnp.concatenate(
        [r[...] for r in atab_refs]
        + [jnp.zeros((_ATOM_PAD - 173, _DIM_H), jnp.float32)], axis=0)
    xb = x_ref[...]
    iota = jax.lax.broadcasted_iota(jnp.int32, (_NB, _ATOM_PAD), 1)
    mh = jnp.zeros((_NB, _ATOM_PAD), jnp.float32)
    off = 0
    for c, d in enumerate(_ATOM_DIMS):
        mh = mh + (iota == xb[:, c:c + 1] + off).astype(jnp.float32)
        off += d
    h = jnp.dot(mh, tbl, preferred_element_type=jnp.float32)
    p = jnp.maximum(jnp.dot(pe_ref[...], w1_ref[...],
                            preferred_element_type=jnp.float32)
                    + b1_ref[...].reshape(1, 32), 0.0)
    p = jnp.maximum(jnp.dot(p, w2_ref[...],
                            preferred_element_type=jnp.float32)
                    + b2_ref[...].reshape(1, 16), 0.0)
    out_ref[...] = jnp.concatenate([h, p], axis=1)


def _sc_edge_body(codes_hbm, c_hbm, out_hbm, codes_v, rows0, rows1, c_sh,
                  sem0, sem1):
    E = codes_hbm.shape[0]
    bpw = E // _NW
    wid = lax.axis_index("s") * 2 + lax.axis_index("c")
    base = wid * bpw

    @pl.when(lax.axis_index("s") == 0)
    def _():
        pltpu.sync_copy(c_hbm, c_sh)

    plsc.subcore_barrier()

    pltpu.sync_copy(codes_hbm.at[pl.ds(base, bpw)], codes_v)

    def pair_body(j, _):
        for b, (rv, sem) in enumerate(((rows0, sem0), (rows1, sem1))):
            c = 2 * j + b
            e0 = base + c * _CH

            @pl.when(j > 0)
            def _():
                pltpu.make_async_copy(
                    rv, out_hbm.at[pl.ds(e0 - 2 * _CH, _CH)], sem).wait()

            pltpu.sync_copy(c_sh.at[codes_v.at[pl.ds(c * _CH, _CH)]], rv)
            pltpu.make_async_copy(rv, out_hbm.at[pl.ds(e0, _CH)], sem).start()
        return 0

    lax.fori_loop(0, _NCH // 2, pair_body, 0)

    e0 = base + (_NCH - 1) * _CH
    pltpu.make_async_copy(rows0, out_hbm.at[pl.ds(e0 - 2 * _CH, _CH)], sem0).wait()
    pltpu.sync_copy(c_sh.at[codes_v.at[pl.ds((_NCH - 1) * _CH, _CH)]], rows0)
    pltpu.make_async_copy(rows0, out_hbm.at[pl.ds(e0, _CH)], sem0).start()

    pltpu.make_async_copy(rows0, out_hbm.at[pl.ds(e0, _CH)], sem0).wait()
    pltpu.make_async_copy(
        rows1, out_hbm.at[pl.ds(e0 - _CH, _CH)], sem1).wait()


def kernel(x, edge_attr, pestat_RWSE, atom_tables, bond_tables, W1, b1, W2, b2):
    N = x.shape[0]
    E = edge_attr.shape[0]

    ea_w = edge_attr.reshape(E // 128, 384)
    codes2d, ctbl = pl.pallas_call(
        _prep_body,
        grid=(1,),
        in_specs=[
            pl.BlockSpec((E // 128, 384), lambda i: (0, 0)),
            pl.BlockSpec((5, _DIM_EMB), lambda i: (0, 0)),
            pl.BlockSpec((6, _DIM_EMB), lambda i: (0, 0)),
            pl.BlockSpec((2, _DIM_EMB), lambda i: (0, 0)),
        ],
        out_specs=[
            pl.BlockSpec((E // 128, 128), lambda i: (0, 0)),
            pl.BlockSpec((_C_ROWS, _DIM_EMB), lambda i: (0, 0)),
        ],
        out_shape=[
            jax.ShapeDtypeStruct((E // 128, 128), jnp.int32),
            jax.ShapeDtypeStruct((_C_ROWS, _DIM_EMB), jnp.float32),
        ],
    )(ea_w, *bond_tables)
    codes = codes2d.reshape(E)

    sc_edge = functools.partial(
        pl.kernel,
        mesh=plsc.VectorSubcoreMesh(core_axis_name="c", subcore_axis_name="s"),
        out_type=jax.ShapeDtypeStruct((E, _DIM_EMB), jnp.float32),
        scratch_types=[
            pltpu.VMEM((E // _NW,), jnp.int32),
            pltpu.VMEM((_CH, _DIM_EMB), jnp.float32),
            pltpu.VMEM((_CH, _DIM_EMB), jnp.float32),
            pltpu.VMEM_SHARED((_C_ROWS, _DIM_EMB), jnp.float32),
            pltpu.SemaphoreType.DMA,
            pltpu.SemaphoreType.DMA,
        ],
    )(_sc_edge_body)
    e = sc_edge(codes, ctbl)

    new_x = pl.pallas_call(
        _node_body,
        grid=(N // _NB,),
        in_specs=[
            pl.BlockSpec((_NB, 9), lambda i: (i, 0)),
            pl.BlockSpec((_NB, 20), lambda i: (i, 0)),
            pl.BlockSpec((20, 32), lambda i: (0, 0)),
            pl.BlockSpec((32,), lambda i: (0,)),
            pl.BlockSpec((32, 16), lambda i: (0, 0)),
            pl.BlockSpec((16,), lambda i: (0,)),
        ] + [pl.BlockSpec((d, _DIM_H), lambda i: (0, 0)) for d in _ATOM_DIMS],
        out_specs=pl.BlockSpec((_NB, _DIM_EMB), lambda i: (i, 0)),
        out_shape=jax.ShapeDtypeStruct((N, _DIM_EMB), jnp.float32),
    )(x, pestat_RWSE, W1, b1, W2, b2, *atom_tables)

    return new_x, e


# speedup vs baseline: 1.1936x; 1.1936x over previous
import functools

import jax
import jax.numpy as jnp
from jax import lax
from jax.experimental import pallas as pl
from jax.experimental.pallas import tpu as pltpu
from jax.experimental.pallas import tpu_sc as plsc

_ATOM_DIMS = (119, 4, 12, 12, 10, 6, 6, 2, 2)
_BOND_DIMS = (5, 6, 2)
_ATOM_PAD = 176
_BOND_PAD = 16
_DIM_H = 112
_DIM_EMB = 128

_NB = 1000
_EB = 16000
_G = _EB // 128
_C_ROWS = 64

_CH = 400
_NW = 32
_NCH = 25


def _prep_body(ea_ref, b0_ref, b1_ref, b2_ref, codes_ref, c_ref):
    widx = jax.lax.broadcasted_iota(jnp.int32, (8, 1), 0)
    w = jnp.where(widx == 0, 12.0,
                  jnp.where(widx == 1, 2.0, jnp.where(widx == 2, 1.0, 0.0)))
    ea = jnp.pad(ea_ref[...].astype(jnp.float32), ((0, 0), (0, 5)))
    code_col = jnp.dot(ea, w, preferred_element_type=jnp.float32)
    codes_ref[pl.ds(pl.program_id(0) * _G, _G), :] = jnp.reshape(
        code_col, (_G, 128)).astype(jnp.int32)

    @pl.when(pl.program_id(0) == 0)
    def _():
        tbl = jnp.concatenate(
            [b0_ref[...], b1_ref[...], b2_ref[...],
             jnp.zeros((_BOND_PAD - 13, _DIM_EMB), jnp.float32)], axis=0)
        k = jax.lax.broadcasted_iota(jnp.int32, (_C_ROWS, 1), 0)
        iota = jax.lax.broadcasted_iota(jnp.int32, (_C_ROWS, _BOND_PAD), 1)
        mh = ((iota == k // 12).astype(jnp.float32)
              + (iota == 5 + (k % 12) // 2).astype(jnp.float32)
              + (iota == 11 + k % 2).astype(jnp.float32))
        c_ref[...] = jnp.dot(mh, tbl, preferred_element_type=jnp.float32)


def _node_body(x_ref, pe_ref, w1_ref, b1_ref, w2_ref, b2_ref, *rest):
    atab_refs, out_ref = rest[:9], rest[9]
    tbl = jnp.concatenate(
        [r[...] for r in atab_refs]
        + [jnp.zeros((_ATOM_PAD - 173, _DIM_H), jnp.float32)], axis=0)
    xb = x_ref[...]
    iota = jax.lax.broadcasted_iota(jnp.int32, (_NB, _ATOM_PAD), 1)
    mh = jnp.zeros((_NB, _ATOM_PAD), jnp.float32)
    off = 0
    for c, d in enumerate(_ATOM_DIMS):
        mh = mh + (iota == xb[:, c:c + 1] + off).astype(jnp.float32)
        off += d
    h = jnp.dot(mh, tbl, preferred_element_type=jnp.float32)
    p = jnp.maximum(jnp.dot(pe_ref[...], w1_ref[...],
                            preferred_element_type=jnp.float32)
                    + b1_ref[...].reshape(1, 32), 0.0)
    p = jnp.maximum(jnp.dot(p, w2_ref[...],
                            preferred_element_type=jnp.float32)
                    + b2_ref[...].reshape(1, 16), 0.0)
    out_ref[...] = jnp.concatenate([h, p], axis=1)


def _sc_edge_body(codes_hbm, c_hbm, out_hbm, codes_v, rows0, rows1, c_sh,
                  sem0, sem1):
    E = codes_hbm.shape[0]
    bpw = E // _NW
    wid = lax.axis_index("s") * 2 + lax.axis_index("c")
    base = wid * bpw

    @pl.when(lax.axis_index("s") == 0)
    def _():
        pltpu.sync_copy(c_hbm, c_sh)

    plsc.subcore_barrier()

    pltpu.sync_copy(codes_hbm.at[pl.ds(base, bpw)], codes_v)

    def pair_body(j, _):
        for b, (rv, sem) in enumerate(((rows0, sem0), (rows1, sem1))):
            c = 2 * j + b
            e0 = base + c * _CH

            @pl.when(j > 0)
            def _():
                pltpu.make_async_copy(
                    rv, out_hbm.at[pl.ds(e0 - 2 * _CH, _CH)], sem).wait()

            pltpu.sync_copy(c_sh.at[codes_v.at[pl.ds(c * _CH, _CH)]], rv)
            pltpu.make_async_copy(rv, out_hbm.at[pl.ds(e0, _CH)], sem).start()
        return 0

    lax.fori_loop(0, _NCH // 2, pair_body, 0)

    e0 = base + (_NCH - 1) * _CH
    pltpu.make_async_copy(rows0, out_hbm.at[pl.ds(e0 - 2 * _CH, _CH)], sem0).wait()
    pltpu.sync_copy(c_sh.at[codes_v.at[pl.ds((_NCH - 1) * _CH, _CH)]], rows0)
    pltpu.make_async_copy(rows0, out_hbm.at[pl.ds(e0, _CH)], sem0).start()

    pltpu.make_async_copy(rows0, out_hbm.at[pl.ds(e0, _CH)], sem0).wait()
    pltpu.make_async_copy(
        rows1, out_hbm.at[pl.ds(e0 - _CH, _CH)], sem1).wait()


def kernel(x, edge_attr, pestat_RWSE, atom_tables, bond_tables, W1, b1, W2, b2):
    N = x.shape[0]
    E = edge_attr.shape[0]

    codes2d, ctbl = pl.pallas_call(
        _prep_body,
        grid=(E // _EB,),
        in_specs=[
            pl.BlockSpec((_EB, 3), lambda i: (i, 0)),
            pl.BlockSpec((5, _DIM_EMB), lambda i: (0, 0)),
            pl.BlockSpec((6, _DIM_EMB), lambda i: (0, 0)),
            pl.BlockSpec((2, _DIM_EMB), lambda i: (0, 0)),
        ],
        out_specs=[
            pl.BlockSpec((E // 128, 128), lambda i: (0, 0)),
            pl.BlockSpec((_C_ROWS, _DIM_EMB), lambda i: (0, 0)),
        ],
        out_shape=[
            jax.ShapeDtypeStruct((E // 128, 128), jnp.int32),
            jax.ShapeDtypeStruct((_C_ROWS, _DIM_EMB), jnp.float32),
        ],
    )(edge_attr, *bond_tables)
    codes = codes2d.reshape(E)

    sc_edge = functools.partial(
        pl.kernel,
        mesh=plsc.VectorSubcoreMesh(core_axis_name="c", subcore_axis_name="s"),
        out_type=jax.ShapeDtypeStruct((E, _DIM_EMB), jnp.float32),
        scratch_types=[
            pltpu.VMEM((E // _NW,), jnp.int32),
            pltpu.VMEM((_CH, _DIM_EMB), jnp.float32),
            pltpu.VMEM((_CH, _DIM_EMB), jnp.float32),
            pltpu.VMEM_SHARED((_C_ROWS, _DIM_EMB), jnp.float32),
            pltpu.SemaphoreType.DMA,
            pltpu.SemaphoreType.DMA,
        ],
    )(_sc_edge_body)
    e = sc_edge(codes, ctbl)

    new_x = pl.pallas_call(
        _node_body,
        grid=(N // _NB,),
        in_specs=[
            pl.BlockSpec((_NB, 9), lambda i: (i, 0)),
            pl.BlockSpec((_NB, 20), lambda i: (i, 0)),
            pl.BlockSpec((20, 32), lambda i: (0, 0)),
            pl.BlockSpec((32,), lambda i: (0,)),
            pl.BlockSpec((32, 16), lambda i: (0, 0)),
            pl.BlockSpec((16,), lambda i: (0,)),
        ] + [pl.BlockSpec((d, _DIM_H), lambda i: (0, 0)) for d in _ATOM_DIMS],
        out_specs=pl.BlockSpec((_NB, _DIM_EMB), lambda i: (i, 0)),
        out_shape=jax.ShapeDtypeStruct((N, _DIM_EMB), jnp.float32),
    )(x, pestat_RWSE, W1, b1, W2, b2, *atom_tables)

    return new_x, e
